# baseline (device time: 47242 ns/iter reference)
import functools

import jax
import jax.numpy as jnp
from jax import lax
from jax.experimental import pallas as pl
from jax.experimental.pallas import tpu as pltpu

N_DEV = 4


def kernel(x, w_mat, scale_x, scale_w):
    k_glob, k_per = x.shape
    _, n = w_mat.shape
    m_per = k_glob // N_DEV

    def body(x_ref, w_ref, sx_ref, sw_ref, out_ref,
             xstage_ref, wstage_ref, xsend_ref, w8_ref, xbuf_ref,
             xload_sems, wload_sems, send_sems, recv_sems):
        me = lax.axis_index("i")

        barrier_sem = pltpu.get_barrier_semaphore()
        for d in range(1, N_DEV):
            pl.semaphore_signal(
                barrier_sem, inc=1,
                device_id=((me + d) % N_DEV,),
                device_id_type=pl.DeviceIdType.MESH,
            )
        pl.semaphore_wait(barrier_sem, N_DEV - 1)

        def x_dma(i, slot):
            t = (me + (i + 1)) % N_DEV if i < N_DEV - 1 else me
            return pltpu.make_async_copy(
                x_ref.at[pl.ds(t * m_per, m_per), :],
                xstage_ref.at[slot],
                xload_sems.at[slot],
            )

        rdmas = []
        x_dma(0, 0).start()
        for i in range(N_DEV):
            slot = i % 2
            x_dma(i, slot).wait()
            if i + 1 < N_DEV:
                x_dma(i + 1, (i + 1) % 2).start()
            xsend_ref[i] = xstage_ref[slot].astype(jnp.float8_e4m3fn)
            if i < N_DEV - 1:
                d = i + 1
                rdma = pltpu.make_async_remote_copy(
                    src_ref=xsend_ref.at[i],
                    dst_ref=xbuf_ref.at[i],
                    send_sem=send_sems.at[i],
                    recv_sem=recv_sems.at[i],
                    device_id=((me + d) % N_DEV,),
                    device_id_type=pl.DeviceIdType.MESH,
                )
                rdma.start()
                rdmas.append(rdma)

        offs = [0, 3, 2, 1]

        def w_dma(idx, slot):
            b = (me + offs[idx]) % N_DEV
            return pltpu.make_async_copy(
                w_ref.at[pl.ds(b * k_per, k_per), :],
                wstage_ref.at[slot],
                wload_sems.at[slot],
            )

        w_dma(0, 0).start()

        for idx in range(N_DEV):
            slot = idx % 2
            w_dma(idx, slot).wait()
            if idx + 1 < N_DEV:
                w_dma(idx + 1, (idx + 1) % 2).start()
            w8_ref[slot] = wstage_ref[slot].astype(jnp.float8_e5m2)
            if idx == 0:
                out_ref[:, :] = jnp.dot(
                    xsend_ref[N_DEV - 1], w8_ref[slot],
                    preferred_element_type=jnp.float32,
                )
            else:
                rdmas[idx - 1].wait_recv()
                out_ref[:, :] += jnp.dot(
                    xbuf_ref[idx - 1], w8_ref[slot],
                    preferred_element_type=jnp.float32,
                )

        for i in range(N_DEV - 1):
            rdmas[i].wait_send()

        scale = sx_ref[0] * sw_ref[0]
        out_ref[:, :] = jnp.maximum(out_ref[:, :] * scale, 0.0)

        @functools.partial(
            pl.run_scoped, exit_sem=pltpu.SemaphoreType.REGULAR
        )
        def _(exit_sem):
            for d in range(1, N_DEV):
                pl.semaphore_signal(
                    exit_sem, inc=1,
                    device_id=((me + d) % N_DEV,),
                    device_id_type=pl.DeviceIdType.MESH,
                )
            pl.semaphore_wait(exit_sem, N_DEV - 1)

    return pl.pallas_call(
        body,
        out_shape=jax.ShapeDtypeStruct((m_per, n), jnp.float32),
        in_specs=[
            pl.BlockSpec(memory_space=pl.ANY),
            pl.BlockSpec(memory_space=pl.ANY),
            pl.BlockSpec(memory_space=pltpu.SMEM),
            pl.BlockSpec(memory_space=pltpu.SMEM),
        ],
        out_specs=pl.BlockSpec(memory_space=pltpu.VMEM),
        scratch_shapes=[
            pltpu.VMEM((2, m_per, k_per), jnp.float32),
            pltpu.VMEM((2, k_per, n), jnp.float32),
            pltpu.VMEM((N_DEV, m_per, k_per), jnp.float8_e4m3fn),
            pltpu.VMEM((2, k_per, n), jnp.float8_e5m2),
            pltpu.VMEM((N_DEV - 1, m_per, k_per), jnp.float8_e4m3fn),
            pltpu.SemaphoreType.DMA((2,)),
            pltpu.SemaphoreType.DMA((2,)),
            pltpu.SemaphoreType.DMA((N_DEV - 1,)),
            pltpu.SemaphoreType.DMA((N_DEV - 1,)),
        ],
        compiler_params=pltpu.CompilerParams(
            collective_id=0, vmem_limit_bytes=52 * 1024 * 1024
        ),
    )(x, w_mat, scale_x, scale_w)
